# baseline (speedup 1.0000x reference)
import functools

import jax
import jax.numpy as jnp
from jax import lax
from jax.experimental import pallas as pl
from jax.experimental.pallas import tpu as pltpu
from jax.experimental.pallas import tpu_sc as plsc

B, T, N, F = 128, 20, 16, 90
NODES = B * N
E = 16384
D_RAW = T * F
D_PAD = 2048
D_HALF = D_PAD // 2
H1, H2 = 128, 256
G1, G2 = 4 * H1, 4 * H2

NC, NS, LANES = 2, 16, 16
HALF = NODES // NC
EDGES_PER_TILE = E // NS
EBATCH = 16
NBATCH = EDGES_PER_TILE // EBATCH
ROWS_PER_TILE = HALF // NS
CHUNKS = D_HALF // 128
CHUNKS_B = 7
DEG_CHUNK = 1024

_HIGH = lax.Precision.HIGHEST


def _prep_body(xt_ref, dst_ref, xpa_ref, xpb_ref, dinv_ref):
    iota = lax.broadcasted_iota(jnp.int32, (NODES, 1), 0)
    deg = jnp.ones((NODES, 1), jnp.float32)
    for i in range(E // DEG_CHUNK):
        d = dst_ref[0:1, i * DEG_CHUNK:(i + 1) * DEG_CHUNK]
        eq = (iota == d).astype(jnp.float32)
        deg = deg + jnp.sum(eq, axis=1, keepdims=True)
    dinv = lax.rsqrt(deg)
    dinv_ref[...] = dinv
    xpa_ref[...] = xt_ref[:, :D_HALF] * dinv
    xpb_ref[...] = xt_ref[:, D_HALF:] * dinv


def _prep(xt, dst2d):
    return pl.pallas_call(
        _prep_body,
        out_shape=(
            jax.ShapeDtypeStruct((NODES, D_HALF), jnp.float32),
            jax.ShapeDtypeStruct((NODES, D_HALF), jnp.float32),
            jax.ShapeDtypeStruct((NODES, 1), jnp.float32),
        ),
    )(xt, dst2d)


def _agg_body(xpa_hbm, xpb_hbm, se_hbm, de_hbm, outa_hbm, outb_hbm,
              sidx, didx, gidx0, soff0, stage0, sem0, ssem0,
              gidx1, soff1, stage1, sem1, ssem1, acc):
    c = lax.axis_index("c")
    s = lax.axis_index("s")
    pltpu.sync_copy(se_hbm.at[pl.ds(s * EDGES_PER_TILE, EDGES_PER_TILE)], sidx)
    pltpu.sync_copy(de_hbm.at[pl.ds(s * EDGES_PER_TILE, EDGES_PER_TILE)], didx)
    lo = c * HALF
    bufs = ((gidx0, soff0, stage0, sem0, ssem0),
            (gidx1, soff1, stage1, sem1, ssem1))

    for xp_hbm, out_hbm, ch in ((xpa_hbm, outa_hbm, CHUNKS),
                                (xpb_hbm, outb_hbm, CHUNKS_B)):
        pltpu.sync_copy(
            xp_hbm.at[pl.ds((lo + s * ROWS_PER_TILE) * CHUNKS,
                            ROWS_PER_TILE * CHUNKS)],
            acc.at[pl.ds(s * ROWS_PER_TILE * CHUNKS, ROWS_PER_TILE * CHUNKS)])
        for gidx, soff, _, _, _ in bufs:
            for j in range(ch, CHUNKS):
                gidx[pl.ds(j * LANES, LANES)] = jnp.full((LANES,), -1,
                                                         jnp.int32)
                soff[pl.ds(j * LANES, LANES)] = jnp.full((LANES,), -1,
                                                         jnp.int32)
        plsc.subcore_barrier()

        def build_issue(buf, b):
            gidx, soff, stage, sem, _ = buf
            off = pl.multiple_of(b * EBATCH, LANES)
            dv = didx[pl.ds(off, LANES)]
            sv = sidx[pl.ds(off, LANES)]
            in_half = (dv >= lo) & (dv < lo + HALF)
            g8 = sv * CHUNKS
            d8 = (dv - lo) * CHUNKS
            for j in range(ch):
                gidx[pl.ds(j * LANES, LANES)] = jnp.where(in_half, g8 + j, -1)
                soff[pl.ds(j * LANES, LANES)] = jnp.where(in_half, d8 + j, -1)
            pltpu.async_copy(
                xp_hbm.at[plsc.Indices(gidx, ignored_value=-1)], stage, sem)

        def gather_done_issue_scatter(buf):
            gidx, soff, stage, sem, ssem = buf
            pltpu.make_async_copy(
                xp_hbm.at[plsc.Indices(gidx, ignored_value=-1)], stage, sem
            ).wait()
            pltpu.async_copy(stage,
                             acc.at[plsc.Indices(soff, ignored_value=-1)],
                             ssem, add=True)

        def scatter_done(buf):
            gidx, soff, stage, sem, ssem = buf
            pltpu.make_async_copy(
                stage, acc.at[plsc.Indices(soff, ignored_value=-1)], ssem
            ).wait()

        build_issue(bufs[0], 0)
        build_issue(bufs[1], 1)

        def body(i, carry):
            b = 2 * i
            gather_done_issue_scatter(bufs[0])
            gather_done_issue_scatter(bufs[1])
            scatter_done(bufs[0])
            build_issue(bufs[0], b + 2)
            scatter_done(bufs[1])
            build_issue(bufs[1], b + 3)
            return carry

        lax.fori_loop(0, NBATCH // 2 - 1, body, 0)
        gather_done_issue_scatter(bufs[0])
        gather_done_issue_scatter(bufs[1])
        scatter_done(bufs[0])
        scatter_done(bufs[1])

        plsc.subcore_barrier()
        pltpu.sync_copy(
            acc.at[pl.ds(s * ROWS_PER_TILE * CHUNKS, ROWS_PER_TILE * CHUNKS)],
            out_hbm.at[pl.ds((lo + s * ROWS_PER_TILE) * CHUNKS,
                             ROWS_PER_TILE * CHUNKS)],
        )


@functools.cache
def _agg_call():
    return pl.kernel(
        _agg_body,
        mesh=plsc.VectorSubcoreMesh(core_axis_name="c", subcore_axis_name="s"),
        out_type=(
            jax.ShapeDtypeStruct((NODES * CHUNKS, 128), jnp.float32),
            jax.ShapeDtypeStruct((NODES * CHUNKS, 128), jnp.float32),
        ),
        scratch_types=[
            pltpu.VMEM((EDGES_PER_TILE,), jnp.int32),
            pltpu.VMEM((EDGES_PER_TILE,), jnp.int32),
            pltpu.VMEM((EBATCH * CHUNKS,), jnp.int32),
            pltpu.VMEM((EBATCH * CHUNKS,), jnp.int32),
            pltpu.VMEM((EBATCH * CHUNKS, 128), jnp.float32),
            pltpu.SemaphoreType.DMA,
            pltpu.SemaphoreType.DMA,
            pltpu.VMEM((EBATCH * CHUNKS,), jnp.int32),
            pltpu.VMEM((EBATCH * CHUNKS,), jnp.int32),
            pltpu.VMEM((EBATCH * CHUNKS, 128), jnp.float32),
            pltpu.SemaphoreType.DMA,
            pltpu.SemaphoreType.DMA,
            pltpu.VMEM_SHARED((HALF * CHUNKS, 128), jnp.float32),
        ],
    )


def _lstm_body(aggt_ref, dvr_ref, wih1_ref, wconv_ref, b1_ref, bc_ref,
               whh1_ref, wih2_ref, b2_ref, whh2_ref,
               wr1_ref, br1_ref, wr2_ref, br2_ref, wr3_ref, br3_ref,
               out_ref, wfold_ref):
    def dot_t(a, b):
        return lax.dot_general(a, b, (((1,), (1,)), ((), ())),
                               preferred_element_type=jnp.float32,
                               precision=_HIGH)

    wsum = jnp.zeros((G1, 256), jnp.float32)
    for n in range(N):
        wn = wih1_ref[:, n * 256:(n + 1) * 256]
        wsum = wsum + wn
        wfold_ref[:, n * F:(n + 1) * F] = dot_t(wn, wconv_ref[...])
    bias1 = b1_ref[...] + dot_t(bc_ref[...], wsum)
    bias2 = b2_ref[...]
    wfold = wfold_ref[...]

    xs = aggt_ref[...] * jnp.tile(dvr_ref[...], (T, 1))
    xproj = dot_t(xs, wfold) + bias1
    wcat = jnp.concatenate([wih2_ref[...], whh2_ref[...]], axis=1)

    h1 = jnp.zeros((B, H1), jnp.float32)
    c1 = jnp.zeros((B, H1), jnp.float32)
    h2 = jnp.zeros((B, H2), jnp.float32)
    c2 = jnp.zeros((B, H2), jnp.float32)
    for t in range(T):
        g = xproj[t * B:(t + 1) * B] + dot_t(h1, whh1_ref[...])
        i = jax.nn.sigmoid(g[:, 0:H1])
        f = jax.nn.sigmoid(g[:, H1:2 * H1])
        gg = jnp.tanh(g[:, 2 * H1:3 * H1])
        o = jax.nn.sigmoid(g[:, 3 * H1:4 * H1])
        c1 = f * c1 + i * gg
        h1 = o * jnp.tanh(c1)

        hcat = jnp.concatenate([h1, h2], axis=1)
        g = dot_t(hcat, wcat) + bias2
        i = jax.nn.sigmoid(g[:, 0:H2])
        f = jax.nn.sigmoid(g[:, H2:2 * H2])
        gg = jnp.tanh(g[:, 2 * H2:3 * H2])
        o = jax.nn.sigmoid(g[:, 3 * H2:4 * H2])
        c2 = f * c2 + i * gg
        h2 = o * jnp.tanh(c2)

    def leaky(v):
        return jnp.where(v >= 0, v, 0.01 * v)

    r = leaky(dot_t(h2, wr1_ref[...]) + br1_ref[...])
    r = leaky(dot_t(r, wr2_ref[...]) + br2_ref[...])
    z = dot_t(r, wr3_ref[...]) + br3_ref[...]
    out_ref[...] = 4.0 * jax.nn.sigmoid(z) + 1.0


def _lstm_head(aggt, dvr, wih1, wconv, b1, bc, whh1, wih2, b2, whh2,
               wr1, br1, wr2, br2, wr3p, br3p):
    return pl.pallas_call(
        _lstm_body,
        out_shape=jax.ShapeDtypeStruct((B, 128), jnp.float32),
        scratch_shapes=[pltpu.VMEM((G1, N * F), jnp.float32)],
    )(aggt.reshape(T * B, N * F), dvr, wih1, wconv, b1, bc, whh1, wih2, b2,
      whh2, wr1, br1, wr2, br2, wr3p, br3p)


def kernel(x, edge_index, W_conv, b_conv, W_ih1, W_hh1, b_ih1, b_hh1,
           W_ih2, W_hh2, b_ih2, b_hh2, W_r1, b_r1, W_r2, b_r2, W_r3, b_r3):
    xt = jnp.transpose(x, (0, 2, 1, 3)).reshape(NODES, D_RAW)
    xt = jnp.pad(xt, ((0, 0), (0, D_PAD - D_RAW)))
    se = edge_index[0]
    de = edge_index[1]

    xpa, xpb, dinv = _prep(xt, de.reshape(1, E))

    xpa8 = xpa.reshape(NODES * CHUNKS, 128)
    xpb8 = xpb.reshape(NODES * CHUNKS, 128)
    outa, outb = _agg_call()(xpa8, xpb8, se, de)
    agg_raw = jnp.concatenate(
        [outa.reshape(NODES, D_HALF), outb.reshape(NODES, D_HALF)], axis=1)

    aggt = (agg_raw[:, :D_RAW].reshape(B, N, T, F)
            .transpose(2, 0, 1, 3).reshape(T, B, N * F))
    dvr = jnp.broadcast_to(dinv.reshape(B, N, 1), (B, N, F)).reshape(B, N * F)

    b1 = (b_ih1 + b_hh1).reshape(1, G1)
    b2 = (b_ih2 + b_hh2).reshape(1, G2)
    bc = b_conv.reshape(1, 256)
    br1 = b_r1.reshape(1, 64)
    br2 = b_r2.reshape(1, 32)
    wr3p = jnp.pad(W_r3, ((0, 123), (0, 0)))
    br3p = jnp.pad(b_r3, (0, 123)).reshape(1, 128)

    out = _lstm_head(aggt, dvr, W_ih1, W_conv, b1, bc, W_hh1, W_ih2, b2,
                     W_hh2, W_r1, br1, W_r2, br2, wr3p, br3p)
    return out[:, :5]


# speedup vs baseline: 1.0839x; 1.0839x over previous
import functools

import jax
import jax.numpy as jnp
from jax import lax
from jax.experimental import pallas as pl
from jax.experimental.pallas import tpu as pltpu
from jax.experimental.pallas import tpu_sc as plsc

B, T, N, F = 128, 20, 16, 90
NODES = B * N
E = 16384
D_RAW = T * F
D_PAD = 2048
D_HALF = D_PAD // 2
H1, H2 = 128, 256
G1, G2 = 4 * H1, 4 * H2

NC, NS, LANES = 2, 16, 16
HALF = NODES // NC
EDGES_PER_TILE = E // NS
EBATCH = 16
NBATCH = EDGES_PER_TILE // EBATCH
ROWS_PER_TILE = HALF // NS
CHUNKS = D_HALF // 128
CHUNKS_B = 7
DEG_CHUNK = 1024

_HIGH = lax.Precision.HIGHEST


def _prep_body(xt_ref, dst_ref, xpa_ref, xpb_ref, dinv_ref):
    iota = lax.broadcasted_iota(jnp.int32, (NODES, 1), 0)
    deg = jnp.ones((NODES, 1), jnp.float32)
    for i in range(E // DEG_CHUNK):
        d = dst_ref[0:1, i * DEG_CHUNK:(i + 1) * DEG_CHUNK]
        eq = (iota == d).astype(jnp.float32)
        deg = deg + jnp.sum(eq, axis=1, keepdims=True)
    dinv = lax.rsqrt(deg)
    dinv_ref[...] = dinv
    xpa_ref[...] = xt_ref[:, :D_HALF] * dinv
    xpb_ref[...] = xt_ref[:, D_HALF:] * dinv


def _prep(xt, dst2d):
    return pl.pallas_call(
        _prep_body,
        out_shape=(
            jax.ShapeDtypeStruct((NODES, D_HALF), jnp.float32),
            jax.ShapeDtypeStruct((NODES, D_HALF), jnp.float32),
            jax.ShapeDtypeStruct((NODES, 1), jnp.float32),
        ),
    )(xt, dst2d)


def _agg_body(xpa_hbm, xpb_hbm, se_hbm, de_hbm, outa_hbm, outb_hbm,
              sidx, didx, gidx0, soff0, stage0, sem0,
              gidx1, soff1, stage1, sem1, acc):
    c = lax.axis_index("c")
    s = lax.axis_index("s")
    pltpu.sync_copy(se_hbm.at[pl.ds(s * EDGES_PER_TILE, EDGES_PER_TILE)], sidx)
    pltpu.sync_copy(de_hbm.at[pl.ds(s * EDGES_PER_TILE, EDGES_PER_TILE)], didx)
    lo = c * HALF
    bufs = ((gidx0, soff0, stage0, sem0),
            (gidx1, soff1, stage1, sem1))

    for xp_hbm, out_hbm, ch in ((xpa_hbm, outa_hbm, CHUNKS),
                                (xpb_hbm, outb_hbm, CHUNKS_B)):
        pltpu.sync_copy(
            xp_hbm.at[pl.ds((lo + s * ROWS_PER_TILE) * CHUNKS,
                            ROWS_PER_TILE * CHUNKS)],
            acc.at[pl.ds(s * ROWS_PER_TILE * CHUNKS, ROWS_PER_TILE * CHUNKS)])
        for gidx, soff, _, _ in bufs:
            for j in range(ch, CHUNKS):
                gidx[pl.ds(j * LANES, LANES)] = jnp.full((LANES,), -1,
                                                         jnp.int32)
                soff[pl.ds(j * LANES, LANES)] = jnp.full((LANES,), -1,
                                                         jnp.int32)
        plsc.subcore_barrier()

        def build_issue(buf, b):
            gidx, soff, stage, sem = buf
            off = pl.multiple_of(b * EBATCH, LANES)
            dv = didx[pl.ds(off, LANES)]
            sv = sidx[pl.ds(off, LANES)]
            in_half = (dv >= lo) & (dv < lo + HALF)
            g8 = sv * CHUNKS
            d8 = (dv - lo) * CHUNKS
            for j in range(ch):
                gidx[pl.ds(j * LANES, LANES)] = jnp.where(in_half, g8 + j, -1)
                soff[pl.ds(j * LANES, LANES)] = jnp.where(in_half, d8 + j, -1)
            pltpu.async_copy(
                xp_hbm.at[plsc.Indices(gidx, ignored_value=-1)], stage, sem)

        def wait_scatter(buf):
            gidx, soff, stage, sem = buf
            pltpu.make_async_copy(
                xp_hbm.at[plsc.Indices(gidx, ignored_value=-1)], stage, sem
            ).wait()
            pltpu.sync_copy(stage,
                            acc.at[plsc.Indices(soff, ignored_value=-1)],
                            add=True)

        build_issue(bufs[0], 0)

        def body(i, carry):
            b = 2 * i
            build_issue(bufs[1], b + 1)
            wait_scatter(bufs[0])
            build_issue(bufs[0], b + 2)
            wait_scatter(bufs[1])
            return carry

        lax.fori_loop(0, NBATCH // 2 - 1, body, 0)
        build_issue(bufs[1], NBATCH - 1)
        wait_scatter(bufs[0])
        wait_scatter(bufs[1])

        plsc.subcore_barrier()
        pltpu.sync_copy(
            acc.at[pl.ds(s * ROWS_PER_TILE * CHUNKS, ROWS_PER_TILE * CHUNKS)],
            out_hbm.at[pl.ds((lo + s * ROWS_PER_TILE) * CHUNKS,
                             ROWS_PER_TILE * CHUNKS)],
        )


@functools.cache
def _agg_call():
    return pl.kernel(
        _agg_body,
        mesh=plsc.VectorSubcoreMesh(core_axis_name="c", subcore_axis_name="s"),
        out_type=(
            jax.ShapeDtypeStruct((NODES * CHUNKS, 128), jnp.float32),
            jax.ShapeDtypeStruct((NODES * CHUNKS, 128), jnp.float32),
        ),
        scratch_types=[
            pltpu.VMEM((EDGES_PER_TILE,), jnp.int32),
            pltpu.VMEM((EDGES_PER_TILE,), jnp.int32),
            pltpu.VMEM((EBATCH * CHUNKS,), jnp.int32),
            pltpu.VMEM((EBATCH * CHUNKS,), jnp.int32),
            pltpu.VMEM((EBATCH * CHUNKS, 128), jnp.float32),
            pltpu.SemaphoreType.DMA,
            pltpu.VMEM((EBATCH * CHUNKS,), jnp.int32),
            pltpu.VMEM((EBATCH * CHUNKS,), jnp.int32),
            pltpu.VMEM((EBATCH * CHUNKS, 128), jnp.float32),
            pltpu.SemaphoreType.DMA,
            pltpu.VMEM_SHARED((HALF * CHUNKS, 128), jnp.float32),
        ],
    )


def _lstm_body(aggt_ref, dvr_ref, wih1_ref, wconv_ref, b1_ref, bc_ref,
               whh1_ref, wih2_ref, b2_ref, whh2_ref,
               wr1_ref, br1_ref, wr2_ref, br2_ref, wr3_ref, br3_ref,
               out_ref, wfold_ref):
    def dot_t(a, b):
        return lax.dot_general(a, b, (((1,), (1,)), ((), ())),
                               preferred_element_type=jnp.float32,
                               precision=_HIGH)

    wsum = jnp.zeros((G1, 256), jnp.float32)
    for n in range(N):
        wn = wih1_ref[:, n * 256:(n + 1) * 256]
        wsum = wsum + wn
        wfold_ref[:, n * F:(n + 1) * F] = dot_t(wn, wconv_ref[...])
    bias1 = b1_ref[...] + dot_t(bc_ref[...], wsum)
    bias2 = b2_ref[...]
    wfold = wfold_ref[...]

    xs = aggt_ref[...] * jnp.tile(dvr_ref[...], (T, 1))
    xproj = dot_t(xs, wfold) + bias1
    wcat = jnp.concatenate([wih2_ref[...], whh2_ref[...]], axis=1)

    h1 = jnp.zeros((B, H1), jnp.float32)
    c1 = jnp.zeros((B, H1), jnp.float32)
    h2 = jnp.zeros((B, H2), jnp.float32)
    c2 = jnp.zeros((B, H2), jnp.float32)
    for t in range(T):
        g = xproj[t * B:(t + 1) * B] + dot_t(h1, whh1_ref[...])
        i = jax.nn.sigmoid(g[:, 0:H1])
        f = jax.nn.sigmoid(g[:, H1:2 * H1])
        gg = jnp.tanh(g[:, 2 * H1:3 * H1])
        o = jax.nn.sigmoid(g[:, 3 * H1:4 * H1])
        c1 = f * c1 + i * gg
        h1 = o * jnp.tanh(c1)

        hcat = jnp.concatenate([h1, h2], axis=1)
        g = dot_t(hcat, wcat) + bias2
        i = jax.nn.sigmoid(g[:, 0:H2])
        f = jax.nn.sigmoid(g[:, H2:2 * H2])
        gg = jnp.tanh(g[:, 2 * H2:3 * H2])
        o = jax.nn.sigmoid(g[:, 3 * H2:4 * H2])
        c2 = f * c2 + i * gg
        h2 = o * jnp.tanh(c2)

    def leaky(v):
        return jnp.where(v >= 0, v, 0.01 * v)

    r = leaky(dot_t(h2, wr1_ref[...]) + br1_ref[...])
    r = leaky(dot_t(r, wr2_ref[...]) + br2_ref[...])
    z = dot_t(r, wr3_ref[...]) + br3_ref[...]
    out_ref[...] = 4.0 * jax.nn.sigmoid(z) + 1.0


def _lstm_head(aggt, dvr, wih1, wconv, b1, bc, whh1, wih2, b2, whh2,
               wr1, br1, wr2, br2, wr3p, br3p):
    return pl.pallas_call(
        _lstm_body,
        out_shape=jax.ShapeDtypeStruct((B, 128), jnp.float32),
        scratch_shapes=[pltpu.VMEM((G1, N * F), jnp.float32)],
    )(aggt.reshape(T * B, N * F), dvr, wih1, wconv, b1, bc, whh1, wih2, b2,
      whh2, wr1, br1, wr2, br2, wr3p, br3p)


def kernel(x, edge_index, W_conv, b_conv, W_ih1, W_hh1, b_ih1, b_hh1,
           W_ih2, W_hh2, b_ih2, b_hh2, W_r1, b_r1, W_r2, b_r2, W_r3, b_r3):
    xt = jnp.transpose(x, (0, 2, 1, 3)).reshape(NODES, D_RAW)
    xt = jnp.pad(xt, ((0, 0), (0, D_PAD - D_RAW)))
    se = edge_index[0]
    de = edge_index[1]

    xpa, xpb, dinv = _prep(xt, de.reshape(1, E))

    xpa8 = xpa.reshape(NODES * CHUNKS, 128)
    xpb8 = xpb.reshape(NODES * CHUNKS, 128)
    outa, outb = _agg_call()(xpa8, xpb8, se, de)
    agg_raw = jnp.concatenate(
        [outa.reshape(NODES, D_HALF), outb.reshape(NODES, D_HALF)], axis=1)

    aggt = (agg_raw[:, :D_RAW].reshape(B, N, T, F)
            .transpose(2, 0, 1, 3).reshape(T, B, N * F))
    dvr = jnp.broadcast_to(dinv.reshape(B, N, 1), (B, N, F)).reshape(B, N * F)

    b1 = (b_ih1 + b_hh1).reshape(1, G1)
    b2 = (b_ih2 + b_hh2).reshape(1, G2)
    bc = b_conv.reshape(1, 256)
    br1 = b_r1.reshape(1, 64)
    br2 = b_r2.reshape(1, 32)
    wr3p = jnp.pad(W_r3, ((0, 123), (0, 0)))
    br3p = jnp.pad(b_r3, (0, 123)).reshape(1, 128)

    out = _lstm_head(aggt, dvr, W_ih1, W_conv, b1, bc, W_hh1, W_ih2, b2,
                     W_hh2, W_r1, br1, W_r2, br2, wr3p, br3p)
    return out[:, :5]
